# baseline (speedup 1.0000x reference)
import functools

import jax
import jax.numpy as jnp
from jax.experimental import pallas as pl
from jax.experimental.pallas import tpu as pltpu


def _mm_body(*refs, nk, bk, k_total, relu, normalize, has_bias):
    if has_bias:
        a_ref, b_ref, bias_ref, o_ref, acc_ref = refs
    else:
        a_ref, b_ref, o_ref, acc_ref = refs
    k = pl.program_id(2)

    @pl.when(k == 0)
    def _():
        acc_ref[...] = jnp.zeros_like(acc_ref)

    a = a_ref[...]
    b = b_ref[...]
    if k_total % bk != 0:
        lim = k_total - k * bk
        col = jax.lax.broadcasted_iota(jnp.int32, a.shape, 1)
        a = jnp.where(col < lim, a, 0.0)
        row = jax.lax.broadcasted_iota(jnp.int32, b.shape, 0)
        b = jnp.where(row < lim, b, 0.0)
    acc_ref[...] += jnp.dot(a, b, preferred_element_type=jnp.float32)

    @pl.when(k == nk - 1)
    def _():
        h = acc_ref[...]
        if has_bias:
            h = h + bias_ref[...]
        if relu:
            h = jnp.maximum(h, 0.0)
        if normalize:
            nrm = jnp.sqrt(jnp.sum(h * h, axis=1, keepdims=True))
            h = h / jnp.maximum(nrm, 1e-12)
        o_ref[...] = h


def _mm(a, b, bias=None, relu=False, normalize=False, bm=1000, bk=512):
    M, K = a.shape
    _, N = b.shape
    bn = N
    nk = -(-K // bk)
    grid = (M // bm, 1, nk)
    in_specs = [
        pl.BlockSpec((bm, bk), lambda m, n, k: (m, k)),
        pl.BlockSpec((bk, bn), lambda m, n, k: (k, n)),
    ]
    args = [a, b]
    if bias is not None:
        in_specs.append(pl.BlockSpec((1, bn), lambda m, n, k: (0, n)))
        args.append(bias.reshape(1, N))
    body = functools.partial(_mm_body, nk=nk, bk=bk, k_total=K, relu=relu,
                             normalize=normalize, has_bias=bias is not None)
    return pl.pallas_call(
        body,
        grid=grid,
        in_specs=in_specs,
        out_specs=pl.BlockSpec((bm, bn), lambda m, n, k: (m, n)),
        out_shape=jax.ShapeDtypeStruct((M, N), jnp.float32),
        scratch_shapes=[pltpu.VMEM((bm, bn), jnp.float32)],
        compiler_params=pltpu.CompilerParams(
            dimension_semantics=("parallel", "parallel", "arbitrary")),
    )(*args)


def kernel(x, adj, W1, b1, W2, b2, W3, b3, W4, b4, W5, b5, W6, b6):
    s1 = _mm(x, W1)
    h1 = _mm(adj, s1, bias=b1, relu=True)
    u2 = _mm(adj, h1)
    h = _mm(u2, W2, bias=b2, relu=True)
    for W, b in ((W3, b3), (W4, b4), (W5, b5)):
        s = _mm(h, W)
        h = _mm(adj, s, bias=b, relu=True)
    s6 = _mm(h, W6)
    return _mm(adj, s6, bias=b6, normalize=True)


# speedup vs baseline: 1.3308x; 1.3308x over previous
import functools

import jax
import jax.numpy as jnp
from jax.experimental import pallas as pl
from jax.experimental.pallas import tpu as pltpu


def _mm_body(*refs, nk, bk, k_total, relu, normalize, has_bias):
    if has_bias:
        a_ref, b_ref, bias_ref, o_ref, acc_ref = refs
    else:
        a_ref, b_ref, o_ref, acc_ref = refs
    k = pl.program_id(2)

    @pl.when(k == 0)
    def _():
        acc_ref[...] = jnp.zeros_like(acc_ref)

    a = a_ref[...]
    b = b_ref[...]
    if k_total % bk != 0:
        lim = k_total - k * bk
        col = jax.lax.broadcasted_iota(jnp.int32, a.shape, 1)
        a = jnp.where(col < lim, a, 0.0)
        row = jax.lax.broadcasted_iota(jnp.int32, b.shape, 0)
        b = jnp.where(row < lim, b, 0.0)
    acc_ref[...] += jnp.dot(a, b, preferred_element_type=jnp.float32)

    @pl.when(k == nk - 1)
    def _():
        h = acc_ref[...]
        if has_bias:
            h = h + bias_ref[...]
        if relu:
            h = jnp.maximum(h, 0.0)
        if normalize:
            nrm = jnp.sqrt(jnp.sum(h * h, axis=1, keepdims=True))
            h = h / jnp.maximum(nrm, 1e-12)
        o_ref[...] = h


def _mm(a, b, bias=None, relu=False, normalize=False, bm=2000, bk=512):
    M, K = a.shape
    _, N = b.shape
    bn = N
    nk = -(-K // bk)
    grid = (M // bm, 1, nk)
    in_specs = [
        pl.BlockSpec((bm, bk), lambda m, n, k: (m, k)),
        pl.BlockSpec((bk, bn), lambda m, n, k: (k, n)),
    ]
    args = [a, b]
    if bias is not None:
        in_specs.append(pl.BlockSpec((1, bn), lambda m, n, k: (0, n)))
        args.append(bias.reshape(1, N))
    body = functools.partial(_mm_body, nk=nk, bk=bk, k_total=K, relu=relu,
                             normalize=normalize, has_bias=bias is not None)
    return pl.pallas_call(
        body,
        grid=grid,
        in_specs=in_specs,
        out_specs=pl.BlockSpec((bm, bn), lambda m, n, k: (m, n)),
        out_shape=jax.ShapeDtypeStruct((M, N), jnp.float32),
        scratch_shapes=[pltpu.VMEM((bm, bn), jnp.float32)],
        compiler_params=pltpu.CompilerParams(
            dimension_semantics=("parallel", "parallel", "arbitrary")),
    )(*args)


def kernel(x, adj, W1, b1, W2, b2, W3, b3, W4, b4, W5, b5, W6, b6):
    s1 = _mm(x, W1)
    h1 = _mm(adj, s1, bias=b1, relu=True)
    u2 = _mm(adj, h1)
    h = _mm(u2, W2, bias=b2, relu=True)
    for W, b in ((W3, b3), (W4, b4), (W5, b5)):
        s = _mm(h, W)
        h = _mm(adj, s, bias=b, relu=True)
    s6 = _mm(h, W6)
    return _mm(adj, s6, bias=b6, normalize=True)
